# baseline (speedup 1.0000x reference)
import functools

import jax
import jax.numpy as jnp
from jax import lax
from jax.experimental import pallas as pl
from jax.experimental.pallas import tpu as pltpu
from jax.experimental.pallas import tpu_sc as plsc

NX = 512
NY = 512
NCH = 64
NVOX = 20000

NC = 2
NS = 16
NW = NC * NS
RANGE = (NX * NY) // NW
RBITS = 13

CPASS = 4
NPAIR = NCH // (2 * CPASS)
CHUNK = 2000
NCHUNK = NVOX // CHUNK
GPC = CHUNK // 16
K = 128
NQ = 8
SSZ = K * NQ
OWNCAP = NVOX + SSZ


_GDN = lax.GatherDimensionNumbers(
    offset_dims=(), collapsed_slice_dims=(0,), start_index_map=(0,))


def _vgather(x, idx):
    return lax.gather(x, idx[:, None], _GDN, slice_sizes=(1,),
                      mode=lax.GatherScatterMode.PROMISE_IN_BOUNDS)


def _body(vf8, c0_hbm, c1_hbm, out_hbm, slabA, slabB, owned, dpk,
          c0b, c1b, gq, featb, gsem, osemA, osemB):
    wid = lax.axis_index("s") * NC + lax.axis_index("c")
    lo = wid * RANGE
    iota = lax.broadcasted_iota(jnp.int32, (16,), 0)
    last15 = iota * 0 + 15
    zeros16 = jnp.zeros((16,), jnp.float32)

    def one_group(gbase, i, cur):
        v0 = c0b[pl.ds(i * 16, 16)]
        v1 = c1b[pl.ds(i * 16, 16)]
        local = (v0 + v1 * NX) - lo
        m = (local >= 0) & (local < RANGE)
        vid = gbase + i * 16 + iota
        packed = (vid << RBITS) | jnp.where(m, local, 0)
        s = plsc.cumsum(m.astype(jnp.int32))
        plsc.store_scatter(owned, [cur + s - 1], packed, mask=m)
        return cur + _vgather(s, last15)

    def chunk_body(g, cur):
        pltpu.sync_copy(c0_hbm.at[pl.ds(g * CHUNK, CHUNK)], c0b)
        pltpu.sync_copy(c1_hbm.at[pl.ds(g * CHUNK, CHUNK)], c1b)

        def grp(i, cur):
            cur = one_group(g * CHUNK, 2 * i, cur)
            return one_group(g * CHUNK, 2 * i + 1, cur)

        cur = lax.fori_loop(0, GPC // 2, grp, cur)
        return one_group(g * CHUNK, GPC - 1, cur)

    nvec = lax.fori_loop(0, NCHUNK, chunk_body, iota * 0)
    n = jnp.max(nvec)
    ng = (n + 15) // 16

    def dedup(j, _):
        pk = owned[pl.ds(j * 16, 16)]
        valid = (j * 16 + iota) < n
        local = pk & (RANGE - 1)
        key2 = (jnp.where(valid, local, RANGE + iota) << 4) | iota
        sk, sv = plsc.sort_key_val(key2, iota)
        skey = sk >> 4
        nxt = _vgather(skey, jnp.minimum(iota + 1, 15))
        m = ((nxt != skey) | (iota == 15)) & (skey < RANGE)
        dpk[pl.ds(j * 16, 16)] = (skey << 5) | (sv << 1) | m.astype(jnp.int32)
        return 0

    lax.fori_loop(0, ng, dedup, 0)

    XPT = RANGE // NY

    def zrow(i, _):
        for c in range(CPASS):
            for r in range(XPT):
                slabA[c, r, pl.ds(i * 16, 16)] = zeros16
                slabB[c, r, pl.ds(i * 16, 16)] = zeros16
        return 0

    lax.fori_loop(0, NY // 16, zrow, 0)

    nss = (n + SSZ - 1) // SSZ

    def run_pass(q):
        def superstep(ss, _):
            sbase = ss * SSZ

            dmas = []
            for t in range(NQ):
                qbase = sbase + t * K

                def gi(j, _, qbase=qbase, t=t):
                    pk = owned[pl.ds(qbase + j * 16, 16)]
                    ok = (qbase + j * 16 + iota) < n
                    gq[t, pl.ds(j * 16, 16)] = jnp.where(
                        ok, (pk >> RBITS) * NPAIR + q, j * 16 + iota)
                    return 0

                lax.fori_loop(0, K // 16, gi, 0)
                dmas.append(pltpu.async_copy(
                    vf8.at[gq.at[t]], featb.at[pl.ds(t * K, K)], gsem))
            for d in dmas:
                d.wait()

            def sc(j, _):
                dp = dpk[pl.ds(sbase + j * 16, 16)]
                m = (dp & 1) == 1
                skey = dp >> 5
                xk = skey >> 9
                yk = skey & (NY - 1)
                sv = (dp >> 1) & 15
                row = j * 16 + sv
                for c in range(CPASS):
                    cvec = iota * 0 + c
                    vals = plsc.load_gather(featb, [row, cvec])
                    plsc.store_scatter(slabA, [cvec, xk, yk], vals, mask=m)
                    vals = plsc.load_gather(featb, [row, cvec + CPASS])
                    plsc.store_scatter(slabB, [cvec, xk, yk], vals, mask=m)
                return 0

            strip = jnp.clip((n - sbase + 15) >> 4, 0, SSZ // 16)
            lax.fori_loop(0, strip, sc, 0)
            return 0

        lax.fori_loop(0, nss, superstep, 0)
        da = pltpu.async_copy(
            slabA,
            out_hbm.at[pl.ds(q * 2 * CPASS, CPASS),
                       pl.ds(wid * XPT, XPT), pl.ds(0, NY)], osemA)
        db = pltpu.async_copy(
            slabB,
            out_hbm.at[pl.ds(q * 2 * CPASS + CPASS, CPASS),
                       pl.ds(wid * XPT, XPT), pl.ds(0, NY)], osemB)
        return da, db

    pend = None
    for q in range(NPAIR):
        if pend is not None:
            pend[0].wait()
            pend[1].wait()
        pend = run_pass(q)
    pend[0].wait()
    pend[1].wait()


@jax.jit
def kernel(voxel_features, coords):
    coords = coords.astype(jnp.int32)
    c0 = coords[:, 0]
    c1 = coords[:, 1]
    vf8 = voxel_features.reshape(NVOX * NPAIR, 2 * CPASS)

    mesh = plsc.VectorSubcoreMesh(core_axis_name="c", subcore_axis_name="s")
    run = functools.partial(
        pl.kernel,
        out_type=jax.ShapeDtypeStruct((NCH, NX, NY), jnp.float32),
        mesh=mesh,
        compiler_params=pltpu.CompilerParams(
            needs_layout_passes=False, use_tc_tiling_on_sc=False),
        scratch_types=[
            pltpu.VMEM((CPASS, NX * NY // NW // NY, NY), jnp.float32),
            pltpu.VMEM((CPASS, NX * NY // NW // NY, NY), jnp.float32),
            pltpu.VMEM((OWNCAP,), jnp.int32),
            pltpu.VMEM((OWNCAP,), jnp.int32),
            pltpu.VMEM((CHUNK,), jnp.int32),
            pltpu.VMEM((CHUNK,), jnp.int32),
            pltpu.VMEM((NQ, K), jnp.int32),
            pltpu.VMEM((SSZ, 2 * CPASS), jnp.float32),
            pltpu.SemaphoreType.DMA,
            pltpu.SemaphoreType.DMA,
            pltpu.SemaphoreType.DMA,
        ],
    )(_body)
    return run(vf8, c0, c1)


# speedup vs baseline: 1.0725x; 1.0725x over previous
import functools

import jax
import jax.numpy as jnp
from jax import lax
from jax.experimental import pallas as pl
from jax.experimental.pallas import tpu as pltpu
from jax.experimental.pallas import tpu_sc as plsc

NX = 512
NY = 512
NCH = 64
NVOX = 20000

NC = 2
NS = 16
NW = NC * NS
RANGE = (NX * NY) // NW
RBITS = 13

CPASS = 4
NPAIR = NCH // (2 * CPASS)
CHUNK = 4000
NCHUNK = NVOX // CHUNK
GPC = CHUNK // 16
K = 128
NQ = 8
SSZ = K * NQ
OWNCAP = NVOX + SSZ


_GDN = lax.GatherDimensionNumbers(
    offset_dims=(), collapsed_slice_dims=(0,), start_index_map=(0,))


def _vgather(x, idx):
    return lax.gather(x, idx[:, None], _GDN, slice_sizes=(1,),
                      mode=lax.GatherScatterMode.PROMISE_IN_BOUNDS)


def _body(vf8, c0_hbm, c1_hbm, out_hbm, slabA, slabB, owned, dpk,
          c0b, c1b, gq, featb, gsem, osemA, osemB):
    wid = lax.axis_index("s") * NC + lax.axis_index("c")
    lo = wid * RANGE
    iota = lax.broadcasted_iota(jnp.int32, (16,), 0)
    last15 = iota * 0 + 15
    zeros16 = jnp.zeros((16,), jnp.float32)

    def prep(gbase, i):
        v0 = c0b[pl.ds(i * 16, 16)]
        v1 = c1b[pl.ds(i * 16, 16)]
        local = (v0 + v1 * NX) - lo
        m = (local >= 0) & (local < RANGE)
        vid = gbase + i * 16 + iota
        return m, (vid << RBITS) | jnp.where(m, local, 0)

    def one_group(gbase, i, cur):
        m, packed = prep(gbase, i)
        s = plsc.cumsum(m.astype(jnp.int32))
        plsc.store_scatter(owned, [cur + s - 1], packed, mask=m)
        return cur + _vgather(s, last15)

    def chunk_body(g, cur):
        pltpu.sync_copy(c0_hbm.at[pl.ds(g * CHUNK, CHUNK)], c0b)
        pltpu.sync_copy(c1_hbm.at[pl.ds(g * CHUNK, CHUNK)], c1b)

        def grp(i, cur):
            m0, p0 = prep(g * CHUNK, 3 * i)
            m1, p1 = prep(g * CHUNK, 3 * i + 1)
            m2, p2 = prep(g * CHUNK, 3 * i + 2)
            s0 = plsc.cumsum(m0.astype(jnp.int32))
            s1 = plsc.cumsum(m1.astype(jnp.int32))
            s2 = plsc.cumsum(m2.astype(jnp.int32))
            t0 = _vgather(s0, last15)
            t1 = _vgather(s1, last15)
            t2 = _vgather(s2, last15)
            plsc.store_scatter(owned, [cur + s0 - 1], p0, mask=m0)
            cur = cur + t0
            plsc.store_scatter(owned, [cur + s1 - 1], p1, mask=m1)
            cur = cur + t1
            plsc.store_scatter(owned, [cur + s2 - 1], p2, mask=m2)
            return cur + t2

        cur = lax.fori_loop(0, GPC // 3, grp, cur)
        return one_group(g * CHUNK, GPC - 1, cur)

    nvec = lax.fori_loop(0, NCHUNK, chunk_body, iota * 0)
    n = jnp.max(nvec)
    ng = (n + 15) // 16

    def dedup(j, _):
        pk = owned[pl.ds(j * 16, 16)]
        valid = (j * 16 + iota) < n
        local = pk & (RANGE - 1)
        key2 = (jnp.where(valid, local, RANGE + iota) << 4) | iota
        sk, sv = plsc.sort_key_val(key2, iota)
        skey = sk >> 4
        nxt = _vgather(skey, jnp.minimum(iota + 1, 15))
        m = ((nxt != skey) | (iota == 15)) & (skey < RANGE)
        dpk[pl.ds(j * 16, 16)] = (skey << 5) | (sv << 1) | m.astype(jnp.int32)
        return 0

    lax.fori_loop(0, ng, dedup, 0)

    XPT = RANGE // NY

    def zrow(i, _):
        for c in range(CPASS):
            for r in range(XPT):
                slabA[c, r, pl.ds(i * 16, 16)] = zeros16
                slabB[c, r, pl.ds(i * 16, 16)] = zeros16
        return 0

    lax.fori_loop(0, NY // 16, zrow, 0)

    nss = (n + SSZ - 1) // SSZ

    def run_pass(q):
        def superstep(ss, _):
            sbase = ss * SSZ

            dmas = []
            for t in range(NQ):
                qbase = sbase + t * K

                def gi(j, _, qbase=qbase, t=t):
                    pk = owned[pl.ds(qbase + j * 16, 16)]
                    ok = (qbase + j * 16 + iota) < n
                    gq[t, pl.ds(j * 16, 16)] = jnp.where(
                        ok, (pk >> RBITS) * NPAIR + q, j * 16 + iota)
                    return 0

                lax.fori_loop(0, K // 16, gi, 0)
                dmas.append(pltpu.async_copy(
                    vf8.at[gq.at[t]], featb.at[pl.ds(t * K, K)], gsem))
            for d in dmas:
                d.wait()

            def sc(j, _):
                dp = dpk[pl.ds(sbase + j * 16, 16)]
                m = (dp & 1) == 1
                skey = dp >> 5
                xk = skey >> 9
                yk = skey & (NY - 1)
                sv = (dp >> 1) & 15
                row = j * 16 + sv
                for c in range(CPASS):
                    cvec = iota * 0 + c
                    vals = plsc.load_gather(featb, [row, cvec])
                    plsc.store_scatter(slabA, [cvec, xk, yk], vals, mask=m)
                    vals = plsc.load_gather(featb, [row, cvec + CPASS])
                    plsc.store_scatter(slabB, [cvec, xk, yk], vals, mask=m)
                return 0

            strip = jnp.clip((n - sbase + 15) >> 4, 0, SSZ // 16)
            lax.fori_loop(0, strip, sc, 0)
            return 0

        lax.fori_loop(0, nss, superstep, 0)
        da = pltpu.async_copy(
            slabA,
            out_hbm.at[pl.ds(q * 2 * CPASS, CPASS),
                       pl.ds(wid * XPT, XPT), pl.ds(0, NY)], osemA)
        db = pltpu.async_copy(
            slabB,
            out_hbm.at[pl.ds(q * 2 * CPASS + CPASS, CPASS),
                       pl.ds(wid * XPT, XPT), pl.ds(0, NY)], osemB)
        return da, db

    pend = None
    for q in range(NPAIR):
        if pend is not None:
            pend[0].wait()
            pend[1].wait()
        pend = run_pass(q)
    pend[0].wait()
    pend[1].wait()


@jax.jit
def kernel(voxel_features, coords):
    coords = coords.astype(jnp.int32)
    c0 = coords[:, 0]
    c1 = coords[:, 1]
    vf8 = voxel_features.reshape(NVOX * NPAIR, 2 * CPASS)

    mesh = plsc.VectorSubcoreMesh(core_axis_name="c", subcore_axis_name="s")
    run = functools.partial(
        pl.kernel,
        out_type=jax.ShapeDtypeStruct((NCH, NX, NY), jnp.float32),
        mesh=mesh,
        compiler_params=pltpu.CompilerParams(
            needs_layout_passes=False, use_tc_tiling_on_sc=False),
        scratch_types=[
            pltpu.VMEM((CPASS, NX * NY // NW // NY, NY), jnp.float32),
            pltpu.VMEM((CPASS, NX * NY // NW // NY, NY), jnp.float32),
            pltpu.VMEM((OWNCAP,), jnp.int32),
            pltpu.VMEM((OWNCAP,), jnp.int32),
            pltpu.VMEM((CHUNK,), jnp.int32),
            pltpu.VMEM((CHUNK,), jnp.int32),
            pltpu.VMEM((NQ, K), jnp.int32),
            pltpu.VMEM((SSZ, 2 * CPASS), jnp.float32),
            pltpu.SemaphoreType.DMA,
            pltpu.SemaphoreType.DMA,
            pltpu.SemaphoreType.DMA,
        ],
    )(_body)
    return run(vf8, c0, c1)


# speedup vs baseline: 1.5359x; 1.4321x over previous
import functools

import jax
import jax.numpy as jnp
from jax import lax
from jax.experimental import pallas as pl
from jax.experimental.pallas import tpu as pltpu
from jax.experimental.pallas import tpu_sc as plsc

NX = 512
NY = 512
NCH = 64
NVOX = 20000

NC = 2
NS = 16
NW = NC * NS
RANGE = (NX * NY) // NW
RBITS = 13

CPASS = 4
NPAIR = NCH // (2 * CPASS)
CHUNK = 4000
NCHUNK = NVOX // CHUNK
GPC = CHUNK // 16
K = 128
NQ = 8
SSZ = K * NQ
OWNCAP = NVOX + SSZ


_GDN = lax.GatherDimensionNumbers(
    offset_dims=(), collapsed_slice_dims=(0,), start_index_map=(0,))


def _vgather(x, idx):
    return lax.gather(x, idx[:, None], _GDN, slice_sizes=(1,),
                      mode=lax.GatherScatterMode.PROMISE_IN_BOUNDS)


def _body(vf8, c0_hbm, c1_hbm, out_hbm, slabA, slabB, owned, dpk,
          c0b, c1b, gq, featb, gsem, osemA, osemB):
    wid = lax.axis_index("s") * NC + lax.axis_index("c")
    lo = wid * RANGE
    iota = lax.broadcasted_iota(jnp.int32, (16,), 0)
    last15 = iota * 0 + 15
    zeros16 = jnp.zeros((16,), jnp.float32)

    def prep(gbase, i):
        v0 = c0b[pl.ds(i * 16, 16)]
        v1 = c1b[pl.ds(i * 16, 16)]
        local = (v0 + v1 * NX) - lo
        m = (local >= 0) & (local < RANGE)
        vid = gbase + i * 16 + iota
        return m, (vid << RBITS) | jnp.where(m, local, 0)

    def one_group(gbase, i, cur):
        m, packed = prep(gbase, i)
        s = plsc.cumsum(m.astype(jnp.int32))
        plsc.store_scatter(owned, [cur + s - 1], packed, mask=m)
        return cur + _vgather(s, last15)

    def chunk_body(g, cur):
        pltpu.sync_copy(c0_hbm.at[pl.ds(g * CHUNK, CHUNK)], c0b)
        pltpu.sync_copy(c1_hbm.at[pl.ds(g * CHUNK, CHUNK)], c1b)

        def grp(i, cur):
            m0, p0 = prep(g * CHUNK, 3 * i)
            m1, p1 = prep(g * CHUNK, 3 * i + 1)
            m2, p2 = prep(g * CHUNK, 3 * i + 2)
            s0 = plsc.cumsum(m0.astype(jnp.int32))
            s1 = plsc.cumsum(m1.astype(jnp.int32))
            s2 = plsc.cumsum(m2.astype(jnp.int32))
            t0 = _vgather(s0, last15)
            t1 = _vgather(s1, last15)
            t2 = _vgather(s2, last15)
            plsc.store_scatter(owned, [cur + s0 - 1], p0, mask=m0)
            cur = cur + t0
            plsc.store_scatter(owned, [cur + s1 - 1], p1, mask=m1)
            cur = cur + t1
            plsc.store_scatter(owned, [cur + s2 - 1], p2, mask=m2)
            return cur + t2

        cur = lax.fori_loop(0, GPC // 3, grp, cur)
        return one_group(g * CHUNK, GPC - 1, cur)

    nvec = lax.fori_loop(0, NCHUNK, chunk_body, iota * 0)
    n = jnp.max(nvec)
    ng = (n + 15) // 16

    def dedup(j, _):
        pk = owned[pl.ds(j * 16, 16)]
        valid = (j * 16 + iota) < n
        local = pk & (RANGE - 1)
        key2 = (jnp.where(valid, local, RANGE + iota) << 4) | iota
        sk, sv = plsc.sort_key_val(key2, iota)
        skey = sk >> 4
        nxt = _vgather(skey, jnp.minimum(iota + 1, 15))
        m = ((nxt != skey) | (iota == 15)) & (skey < RANGE)
        dpk[pl.ds(j * 16, 16)] = (skey << 5) | (sv << 1) | m.astype(jnp.int32)
        return 0

    lax.fori_loop(0, ng, dedup, 0)

    def zrow(i, _):
        for c in range(CPASS):
            slabA[c, pl.ds(i * 16, 16)] = zeros16
            slabB[c, pl.ds(i * 16, 16)] = zeros16
        return 0

    lax.fori_loop(0, RANGE // 16, zrow, 0)

    nss = (n + SSZ - 1) // SSZ

    def run_pass(q):
        def superstep(ss, _):
            sbase = ss * SSZ

            dmas = []
            for t in range(NQ):
                qbase = sbase + t * K

                def gi(j, _, qbase=qbase, t=t):
                    pk = owned[pl.ds(qbase + j * 16, 16)]
                    ok = (qbase + j * 16 + iota) < n
                    gq[t, pl.ds(j * 16, 16)] = jnp.where(
                        ok, (pk >> RBITS) * NPAIR + q, j * 16 + iota)
                    return 0

                lax.fori_loop(0, K // 16, gi, 0)
                dmas.append(pltpu.async_copy(
                    vf8.at[gq.at[t]], featb.at[pl.ds(t * K, K)], gsem))
            for d in dmas:
                d.wait()

            def sc(j, _):
                dp = dpk[pl.ds(sbase + j * 16, 16)]
                m = (dp & 1) == 1
                skey = dp >> 5
                xk = skey >> 9
                yk = skey & (NY - 1)
                toff = (((xk >> 3) << 12) | ((yk >> 7) << 10)
                        | ((xk & 7) << 7) | (yk & 127))
                sv = (dp >> 1) & 15
                row = j * 16 + sv
                for c in range(CPASS):
                    cvec = iota * 0 + c
                    vals = plsc.load_gather(featb, [row, cvec])
                    plsc.store_scatter(slabA, [cvec, toff], vals, mask=m)
                    vals = plsc.load_gather(featb, [row, cvec + CPASS])
                    plsc.store_scatter(slabB, [cvec, toff], vals, mask=m)
                return 0

            strip = jnp.clip((n - sbase + 15) >> 4, 0, SSZ // 16)
            lax.fori_loop(0, strip, sc, 0)
            return 0

        lax.fori_loop(0, nss, superstep, 0)
        da = pltpu.async_copy(
            slabA,
            out_hbm.at[pl.ds(q * 2 * CPASS, CPASS), pl.ds(lo, RANGE)], osemA)
        db = pltpu.async_copy(
            slabB,
            out_hbm.at[pl.ds(q * 2 * CPASS + CPASS, CPASS), pl.ds(lo, RANGE)],
            osemB)
        return da, db

    pend = None
    for q in range(NPAIR):
        if pend is not None:
            pend[0].wait()
            pend[1].wait()
        pend = run_pass(q)
    pend[0].wait()
    pend[1].wait()


@jax.jit
def kernel(voxel_features, coords):
    coords = coords.astype(jnp.int32)
    c0 = coords[:, 0]
    c1 = coords[:, 1]
    vf8 = voxel_features.reshape(NVOX * NPAIR, 2 * CPASS)

    mesh = plsc.VectorSubcoreMesh(core_axis_name="c", subcore_axis_name="s")
    run = functools.partial(
        pl.kernel,
        out_type=jax.ShapeDtypeStruct((NCH, NX * NY), jnp.float32),
        mesh=mesh,
        compiler_params=pltpu.CompilerParams(
            needs_layout_passes=False, use_tc_tiling_on_sc=False),
        scratch_types=[
            pltpu.VMEM((CPASS, RANGE), jnp.float32),
            pltpu.VMEM((CPASS, RANGE), jnp.float32),
            pltpu.VMEM((OWNCAP,), jnp.int32),
            pltpu.VMEM((OWNCAP,), jnp.int32),
            pltpu.VMEM((CHUNK,), jnp.int32),
            pltpu.VMEM((CHUNK,), jnp.int32),
            pltpu.VMEM((NQ, K), jnp.int32),
            pltpu.VMEM((SSZ, 2 * CPASS), jnp.float32),
            pltpu.SemaphoreType.DMA,
            pltpu.SemaphoreType.DMA,
            pltpu.SemaphoreType.DMA,
        ],
    )(_body)
    canvas = run(vf8, c0, c1)
    canvas = canvas.reshape(NCH, NX // 8, NY // 128, 8, 128)
    canvas = canvas.transpose(0, 1, 3, 2, 4)
    return canvas.reshape(NCH, NX, NY)


# speedup vs baseline: 1.5413x; 1.0035x over previous
import functools

import jax
import jax.numpy as jnp
from jax import lax
from jax.experimental import pallas as pl
from jax.experimental.pallas import tpu as pltpu
from jax.experimental.pallas import tpu_sc as plsc

NX = 512
NY = 512
NCH = 64
NVOX = 20000

NC = 2
NS = 16
NW = NC * NS
RANGE = (NX * NY) // NW
RBITS = 13

CPASS = 4
NPAIR = NCH // (2 * CPASS)
CHUNK = 4000
NCHUNK = NVOX // CHUNK
GPC = CHUNK // 16
K = 128
NQ = 8
SSZ = K * NQ
OWNCAP = NVOX + SSZ


_GDN = lax.GatherDimensionNumbers(
    offset_dims=(), collapsed_slice_dims=(0,), start_index_map=(0,))


def _vgather(x, idx):
    return lax.gather(x, idx[:, None], _GDN, slice_sizes=(1,),
                      mode=lax.GatherScatterMode.PROMISE_IN_BOUNDS)


def _body(vf8, c0_hbm, c1_hbm, out_hbm, slabA, slabB, owned, dpk,
          c0b, c1b, gq, featb, gsem, osemA, osemB):
    wid = lax.axis_index("s") * NC + lax.axis_index("c")
    lo = wid * RANGE
    iota = lax.broadcasted_iota(jnp.int32, (16,), 0)
    last15 = iota * 0 + 15
    zeros16 = jnp.zeros((16,), jnp.float32)

    def prep(gbase, i):
        v0 = c0b[pl.ds(i * 16, 16)]
        v1 = c1b[pl.ds(i * 16, 16)]
        local = (v0 + v1 * NX) - lo
        m = local.astype(jnp.uint32) < RANGE
        vid = gbase + i * 16 + iota
        return m, (vid << RBITS) | (local & (RANGE - 1))

    def one_group(gbase, i, cur):
        m, packed = prep(gbase, i)
        s = plsc.cumsum(m.astype(jnp.int32))
        plsc.store_scatter(owned, [cur + s - 1], packed, mask=m)
        return cur + _vgather(s, last15)

    def chunk_body(g, cur):
        pltpu.sync_copy(c0_hbm.at[pl.ds(g * CHUNK, CHUNK)], c0b)
        pltpu.sync_copy(c1_hbm.at[pl.ds(g * CHUNK, CHUNK)], c1b)

        def grp(i, cur):
            m0, p0 = prep(g * CHUNK, 3 * i)
            m1, p1 = prep(g * CHUNK, 3 * i + 1)
            m2, p2 = prep(g * CHUNK, 3 * i + 2)
            s0 = plsc.cumsum(m0.astype(jnp.int32))
            s1 = plsc.cumsum(m1.astype(jnp.int32))
            s2 = plsc.cumsum(m2.astype(jnp.int32))
            t0 = _vgather(s0, last15)
            t1 = _vgather(s1, last15)
            t2 = _vgather(s2, last15)
            plsc.store_scatter(owned, [cur + s0 - 1], p0, mask=m0)
            cur = cur + t0
            plsc.store_scatter(owned, [cur + s1 - 1], p1, mask=m1)
            cur = cur + t1
            plsc.store_scatter(owned, [cur + s2 - 1], p2, mask=m2)
            return cur + t2

        cur = lax.fori_loop(0, GPC // 3, grp, cur)
        return one_group(g * CHUNK, GPC - 1, cur)

    nvec = lax.fori_loop(0, NCHUNK, chunk_body, iota * 0)
    n = jnp.max(nvec)
    ng = (n + 15) // 16

    def dedup(j, _):
        pk = owned[pl.ds(j * 16, 16)]
        valid = (j * 16 + iota) < n
        local = pk & (RANGE - 1)
        key2 = (jnp.where(valid, local, RANGE + iota) << 4) | iota
        sk, sv = plsc.sort_key_val(key2, iota)
        skey = sk >> 4
        nxt = _vgather(skey, jnp.minimum(iota + 1, 15))
        m = ((nxt != skey) | (iota == 15)) & (skey < RANGE)
        xk = skey >> 9
        yk = skey & (NY - 1)
        toff = (((xk >> 3) << 12) | ((yk >> 7) << 10)
                | ((xk & 7) << 7) | (yk & 127))
        dpk[pl.ds(j * 16, 16)] = (toff << 5) | (sv << 1) | m.astype(jnp.int32)
        return 0

    lax.fori_loop(0, ng, dedup, 0)

    def zrow(i, _):
        for c in range(CPASS):
            slabA[c, pl.ds(i * 16, 16)] = zeros16
            slabB[c, pl.ds(i * 16, 16)] = zeros16
        return 0

    lax.fori_loop(0, RANGE // 16, zrow, 0)

    nss = (n + SSZ - 1) // SSZ

    def run_pass(q):
        def superstep(ss, _):
            sbase = ss * SSZ

            dmas = []
            for t in range(NQ):
                qbase = sbase + t * K

                def gi(j, _, qbase=qbase, t=t):
                    pk = owned[pl.ds(qbase + j * 16, 16)]
                    ok = (qbase + j * 16 + iota) < n
                    gq[t, pl.ds(j * 16, 16)] = jnp.where(
                        ok, (pk >> RBITS) * NPAIR + q, j * 16 + iota)
                    return 0

                lax.fori_loop(0, K // 16, gi, 0)
                dmas.append(pltpu.async_copy(
                    vf8.at[gq.at[t]], featb.at[pl.ds(t * K, K)], gsem))
            for d in dmas:
                d.wait()

            def sc(j, _):
                dp = dpk[pl.ds(sbase + j * 16, 16)]
                m = (dp & 1) == 1
                toff = dp >> 5
                sv = (dp >> 1) & 15
                row = j * 16 + sv
                for c in range(CPASS):
                    cvec = iota * 0 + c
                    vals = plsc.load_gather(featb, [row, cvec])
                    plsc.store_scatter(slabA, [cvec, toff], vals, mask=m)
                    vals = plsc.load_gather(featb, [row, cvec + CPASS])
                    plsc.store_scatter(slabB, [cvec, toff], vals, mask=m)
                return 0

            strip = jnp.clip((n - sbase + 15) >> 4, 0, SSZ // 16)
            lax.fori_loop(0, strip, sc, 0)
            return 0

        lax.fori_loop(0, nss, superstep, 0)
        da = pltpu.async_copy(
            slabA,
            out_hbm.at[pl.ds(q * 2 * CPASS, CPASS), pl.ds(lo, RANGE)], osemA)
        db = pltpu.async_copy(
            slabB,
            out_hbm.at[pl.ds(q * 2 * CPASS + CPASS, CPASS), pl.ds(lo, RANGE)],
            osemB)
        return da, db

    pend = None
    for q in range(NPAIR):
        if pend is not None:
            pend[0].wait()
            pend[1].wait()
        pend = run_pass(q)
    pend[0].wait()
    pend[1].wait()


@jax.jit
def kernel(voxel_features, coords):
    coords = coords.astype(jnp.int32)
    c0 = coords[:, 0]
    c1 = coords[:, 1]
    vf8 = voxel_features.reshape(NVOX * NPAIR, 2 * CPASS)

    mesh = plsc.VectorSubcoreMesh(core_axis_name="c", subcore_axis_name="s")
    run = functools.partial(
        pl.kernel,
        out_type=jax.ShapeDtypeStruct((NCH, NX * NY), jnp.float32),
        mesh=mesh,
        compiler_params=pltpu.CompilerParams(
            needs_layout_passes=False, use_tc_tiling_on_sc=False),
        scratch_types=[
            pltpu.VMEM((CPASS, RANGE), jnp.float32),
            pltpu.VMEM((CPASS, RANGE), jnp.float32),
            pltpu.VMEM((OWNCAP,), jnp.int32),
            pltpu.VMEM((OWNCAP,), jnp.int32),
            pltpu.VMEM((CHUNK,), jnp.int32),
            pltpu.VMEM((CHUNK,), jnp.int32),
            pltpu.VMEM((NQ, K), jnp.int32),
            pltpu.VMEM((SSZ, 2 * CPASS), jnp.float32),
            pltpu.SemaphoreType.DMA,
            pltpu.SemaphoreType.DMA,
            pltpu.SemaphoreType.DMA,
        ],
    )(_body)
    canvas = run(vf8, c0, c1)
    canvas = canvas.reshape(NCH, NX // 8, NY // 128, 8, 128)
    canvas = canvas.transpose(0, 1, 3, 2, 4)
    return canvas.reshape(NCH, NX, NY)
